# baseline (speedup 1.0000x reference)
import functools

import jax
import jax.numpy as jnp
from jax import lax
from jax.experimental import pallas as pl
from jax.experimental.pallas import tpu as pltpu
from jax.experimental.pallas import tpu_sc as plsc

VOCAB = 1000000
D_MODEL = 64
BATCH = 4096
HIST = 200

NUM_CORES = 2
NUM_SUBCORES = 16
NUM_WORKERS = NUM_CORES * NUM_SUBCORES

B_TOTAL = BATCH * HIST
PER_W = B_TOTAL // NUM_WORKERS
CHUNK = 128
NUM_CHUNKS = PER_W // CHUNK

_mesh = plsc.VectorSubcoreMesh(core_axis_name="c", subcore_axis_name="s")


@functools.partial(
    pl.kernel,
    out_type=jax.ShapeDtypeStruct((B_TOTAL, D_MODEL), jnp.float32),
    mesh=_mesh,
    scratch_types=[
        [pltpu.VMEM((CHUNK,), jnp.int32)] * 2,
        [pltpu.VMEM((CHUNK,), jnp.int32)] * 2,
        pltpu.VMEM((2, CHUNK, 2 * D_MODEL), jnp.float32),
        pltpu.VMEM((2, CHUNK, D_MODEL), jnp.float32),
        [pltpu.SemaphoreType.DMA] * 2,
        [pltpu.SemaphoreType.DMA] * 2,
    ],
)
def _embed(idx_hbm, tabp_hbm, out_hbm, ridx, gidx, pbuf, obuf, gsems, osems):
    wid = lax.axis_index("s") * NUM_CORES + lax.axis_index("c")
    base0 = pl.multiple_of(wid * PER_W, 8)

    def gstart(c, b):
        base = pl.multiple_of(base0 + c * CHUNK, 8)
        pltpu.sync_copy(idx_hbm.at[pl.ds(base, CHUNK)], ridx[b])
        for m in range(CHUNK // 16):
            sl = pl.ds(m * 16, 16)
            gidx[b][sl] = lax.shift_right_logical(ridx[b][sl], 1)
        pltpu.async_copy(tabp_hbm.at[gidx[b]], pbuf.at[b], gsems[b])

    def gwait(b):
        pltpu.make_async_copy(
            tabp_hbm.at[gidx[b]], pbuf.at[b], gsems[b]
        ).wait()

    def ostart(c, b):
        base = pl.multiple_of(base0 + c * CHUNK, 8)
        pltpu.async_copy(obuf.at[b], out_hbm.at[pl.ds(base, CHUNK)], osems[b])

    def owait(b):
        pltpu.make_async_copy(
            obuf.at[b], out_hbm.at[pl.ds(base0, CHUNK)], osems[b]
        ).wait()

    def compact_rows(b):
        @pl.loop(0, CHUNK // 16)
        def _(q):
            vv = ridx[b][pl.ds(q * 16, 16)]
            offs = (vv & 1) * D_MODEL
            for t in range(16):
                k = q * 16 + t
                ofs = offs[t]
                for m in range(D_MODEL // 16):
                    obuf[b, k, pl.ds(m * 16, 16)] = pbuf[
                        b, k, pl.ds(ofs + m * 16, 16)
                    ]

    gstart(0, 0)
    gstart(1, 1)

    def gpair(j, carry):
        for b in range(2):
            c = 2 * j + b
            gwait(b)

            @pl.when(c >= 2)
            def _():
                owait(b)

            compact_rows(b)
            ostart(c, b)

            @pl.when(c + 2 < NUM_CHUNKS)
            def _():
                gstart(c + 2, b)

        return carry

    lax.fori_loop(0, NUM_CHUNKS // 2, gpair, 0)
    for b in range(2):
        owait(b)


def kernel(X, table):
    idx = X.reshape(-1)
    tabp = table.reshape(VOCAB // 2, 2 * D_MODEL)
    out = _embed(idx, tabp)
    return out.reshape(X.shape + (table.shape[1],))
